# baseline (speedup 1.0000x reference)
import functools

import jax
import jax.numpy as jnp
from jax import lax
from jax.experimental import pallas as pl
from jax.experimental.pallas import tpu as pltpu
from jax.experimental.pallas import tpu_sc as plsc

N = 10000
E = 320000
HID = 128
OUT_DIM = 40
NUM_LAYERS = 7
EPS = 1e-7
BN_EPS = 1e-5

NT = 16
EB = 112
CH = 3
NJ = -(-E // (NT * EB * CH))
NJ = 3 * (-(-NJ // 3))
NB = NJ * CH
E_PAD = NT * NB * EB
ACC_ROWS = 10240
ZR = ACC_ROWS // NT
RB = 2000


def _sc_aggregate(tg, tp, ix_p, zeros_blk):
    mesh = plsc.VectorSubcoreMesh(core_axis_name="c", subcore_axis_name="s")

    @functools.partial(
        pl.kernel,
        out_type=(jax.ShapeDtypeStruct((ACC_ROWS, HID), jnp.float32),
                  jax.ShapeDtypeStruct((ACC_ROWS, HID), jnp.float32)),
        mesh=mesh,
        scratch_types=(
            [pltpu.VMEM((2, CH, EB), jnp.int32)] * 3
            + [pltpu.VMEM((EB, HID), jnp.float32)] * 3
            + [pltpu.VMEM_SHARED((ACC_ROWS, HID), jnp.float32)]
            + [pltpu.SemaphoreType.DMA] * 9
        ),
    )
    def k(tg_hbm, tp_hbm, ix_hbm, z_hbm, den_out, num_out,
          ib0, ib1, ib2, rows0, rows1, rows2, acc,
          csem0, csem1, csem2, gsem0, gsem1, gsem2, ssem0, ssem1, ssem2):
        c = lax.axis_index("c")
        s = lax.axis_index("s")

        IB = [ib0, ib1, ib2]
        RW = [rows0, rows1, rows2]
        CS = [csem0, csem1, csem2]
        GS = [gsem0, gsem1, gsem2]
        SS = [ssem0, ssem1, ssem2]

        pltpu.sync_copy(z_hbm, acc.at[pl.ds(s * ZR, ZR)])
        plsc.subcore_barrier()

        def run(t_hbm):
            def chunk_fetch(kc, q):
                pltpu.async_copy(ix_hbm.at[s, kc], IB[q], CS[q])

            def chunk_wait(kc, q):
                pltpu.make_async_copy(ix_hbm.at[s, kc], IB[q], CS[q]).wait()

            def gather_start(r, sref):
                pltpu.async_copy(t_hbm.at[sref], RW[r], GS[r])

            def gather_wait(r, sref):
                pltpu.make_async_copy(t_hbm.at[sref], RW[r], GS[r]).wait()

            def scatter_start(r, dref):
                pltpu.async_copy(RW[r], acc.at[dref], SS[r], add=True)

            def scatter_wait(r, dref):
                pltpu.make_async_copy(RW[r], acc.at[dref], SS[r]).wait()

            def visit(kc, q, v, first=False, fetch=True, wait_next=True,
                      gnext=True):
                r = v
                rp = (v + 2) % 3
                qn = (q + 1) % 3
                qp = (q + 2) % 3
                gather_wait(r, IB[q].at[0, v])
                if not first:
                    if v == 0:
                        scatter_wait(rp, IB[qp].at[1, 2])
                    else:
                        scatter_wait(rp, IB[q].at[1, v - 1])
                if v == 0 and fetch:
                    chunk_fetch(kc + 2, qp)
                if v == 1 and wait_next:
                    chunk_wait(kc + 1, qn)
                scatter_start(r, IB[q].at[1, v])
                if gnext:
                    if v == 0:
                        gather_start(rp, IB[q].at[0, 2])
                    else:
                        gather_start(rp, IB[qn].at[0, v - 1])

            chunk_fetch(0, 0)
            chunk_fetch(1, 1)
            chunk_wait(0, 0)
            gather_start(0, IB[0].at[0, 0])
            gather_start(1, IB[0].at[0, 1])

            visit(0, 0, 0, first=True)
            visit(0, 0, 1)
            visit(0, 0, 2)
            for kk in (1, 2):
                visit(kk, kk, 0)
                visit(kk, kk, 1)
                visit(kk, kk, 2)

            def body(m, carry):
                k0 = 3 * m
                for q in (0, 1, 2):
                    visit(k0 + q, q, 0)
                    visit(k0 + q, q, 1)
                    visit(k0 + q, q, 2)
                return carry

            lax.fori_loop(1, NJ // 3 - 1, body, 0)

            visit(NJ - 3, 0, 0)
            visit(NJ - 3, 0, 1)
            visit(NJ - 3, 0, 2)
            visit(NJ - 2, 1, 0, fetch=False)
            visit(NJ - 2, 1, 1)
            visit(NJ - 2, 1, 2)
            visit(NJ - 1, 2, 0, fetch=False)
            visit(NJ - 1, 2, 1, wait_next=False, gnext=False)
            visit(NJ - 1, 2, 2, gnext=False)
            scatter_wait(2, IB[2].at[1, 2])

        @pl.when(c == 0)
        def _():
            run(tg_hbm)

        @pl.when(c == 1)
        def _():
            run(tp_hbm)

        plsc.subcore_barrier()

        @pl.when(c == 0)
        def _():
            pltpu.sync_copy(acc.at[pl.ds(s * ZR, ZR)],
                            den_out.at[pl.ds(s * ZR, ZR)])

        @pl.when(c == 1)
        def _():
            pltpu.sync_copy(acc.at[pl.ds(s * ZR, ZR)],
                            num_out.at[pl.ds(s * ZR, ZR)])

    den, num = k(tg, tp, ix_p, zeros_blk)
    return den[:N], num[:N]


def _enc_body(x_ref, w_ref, b_ref, o_ref):
    o_ref[...] = jnp.dot(x_ref[...], w_ref[...],
                         preferred_element_type=jnp.float32) + b_ref[...]


def _encoder(x, w, b):
    return pl.pallas_call(
        _enc_body,
        grid=(N // RB,),
        in_specs=[pl.BlockSpec((RB, HID), lambda i: (i, 0)),
                  pl.BlockSpec((HID, HID), lambda i: (0, 0)),
                  pl.BlockSpec((1, HID), lambda i: (0, 0))],
        out_specs=pl.BlockSpec((RB, HID), lambda i: (i, 0)),
        out_shape=jax.ShapeDtypeStruct((N, HID), jnp.float32),
    )(x, w, b)


def _pre_body(h_ref, g_ref, b_ref, hn_ref, tg_ref, tp_ref):
    h = h_ref[...]
    mean = jnp.mean(h, axis=0, keepdims=True)
    d = h - mean
    var = jnp.mean(d * d, axis=0, keepdims=True)
    hn = d * lax.rsqrt(var + BN_EPS) * g_ref[...] + b_ref[...]
    hn = jnp.maximum(hn, 0.0)
    g = hn + EPS
    cmax = jnp.max(g, axis=0, keepdims=True)
    tg = jnp.exp(g - cmax)
    hn_ref[...] = hn
    tg_ref[...] = tg
    tp_ref[...] = g * tg


def _pre(h, gamma, beta):
    spec = pl.BlockSpec((N, HID), lambda: (0, 0))
    vspec = pl.BlockSpec((1, HID), lambda: (0, 0))
    return pl.pallas_call(
        _pre_body,
        in_specs=[spec, vspec, vspec],
        out_specs=(spec, spec, spec),
        out_shape=(jax.ShapeDtypeStruct((N, HID), jnp.float32),) * 3,
    )(h, gamma, beta)


def _post_body(hn_ref, num_ref, den_ref, h_ref, w_ref, b_ref, o_ref):
    msg = num_ref[...] / (den_ref[...] + 1e-16)
    l2 = jnp.sqrt(jnp.sum(msg * msg, axis=1, keepdims=True))
    hn = hn_ref[...]
    fn = jnp.sqrt(jnp.sum(hn * hn, axis=1, keepdims=True))
    msg = msg / jnp.maximum(l2, 1e-12) * fn
    feats = hn + msg
    o_ref[...] = h_ref[...] + jnp.dot(feats, w_ref[...],
                                      preferred_element_type=jnp.float32) + b_ref[...]


def _post(hn, num, den, h, w, b):
    rspec = pl.BlockSpec((RB, HID), lambda i: (i, 0))
    return pl.pallas_call(
        _post_body,
        grid=(N // RB,),
        in_specs=[rspec, rspec, rspec, rspec,
                  pl.BlockSpec((HID, HID), lambda i: (0, 0)),
                  pl.BlockSpec((1, HID), lambda i: (0, 0))],
        out_specs=rspec,
        out_shape=jax.ShapeDtypeStruct((N, HID), jnp.float32),
    )(hn, num, den, h, w, b)


def _final_body(h_ref, w_ref, b_ref, o_ref):
    z = jnp.dot(h_ref[...], w_ref[...],
                preferred_element_type=jnp.float32) + b_ref[...]
    col = lax.broadcasted_iota(jnp.int32, z.shape, 1)
    valid = col < OUT_DIM
    zm = jnp.where(valid, z, -jnp.inf)
    mx = jnp.max(zm, axis=1, keepdims=True)
    e = jnp.where(valid, jnp.exp(zm - mx), 0.0)
    lse = jnp.log(jnp.sum(e, axis=1, keepdims=True))
    out = zm - mx - lse
    o_ref[...] = out[:, :OUT_DIM]


def _final(h, wp, bp):
    return pl.pallas_call(
        _final_body,
        grid=(N // RB,),
        in_specs=[pl.BlockSpec((RB, HID), lambda i: (i, 0)),
                  pl.BlockSpec((HID, HID), lambda i: (0, 0)),
                  pl.BlockSpec((1, HID), lambda i: (0, 0))],
        out_specs=pl.BlockSpec((RB, OUT_DIM), lambda i: (i, 0)),
        out_shape=jax.ShapeDtypeStruct((N, OUT_DIM), jnp.float32),
    )(h, wp, bp)


def kernel(node_feats, edge_index, W_enc, b_enc, W_conv, b_conv,
           bn_gamma, bn_beta, W_out, b_out):
    src = edge_index[0]
    dst = edge_index[1]
    pad = E_PAD - E
    src_p = jnp.concatenate([src, jnp.zeros((pad,), jnp.int32)]).reshape(
        NT, NJ, CH, EB)
    dst_p = jnp.concatenate([dst, jnp.full((pad,), N, jnp.int32)]).reshape(
        NT, NJ, CH, EB)
    ix_p = jnp.stack([src_p, dst_p], axis=2)
    zeros_blk = jnp.zeros((ZR, HID), jnp.float32)

    h = _encoder(node_feats, W_enc, b_enc.reshape(1, HID))
    for i in range(NUM_LAYERS):
        hn, tg, tp = _pre(h, bn_gamma[i].reshape(1, HID),
                          bn_beta[i].reshape(1, HID))
        den, num = _sc_aggregate(tg, tp, ix_p, zeros_blk)
        h = _post(hn, num, den, h, W_conv[i], b_conv[i].reshape(1, HID))

    wp = jnp.pad(W_out, ((0, 0), (0, HID - OUT_DIM)))
    bp = jnp.pad(b_out, (0, HID - OUT_DIM)).reshape(1, HID)
    return _final(h, wp, bp)


# speedup vs baseline: 1.0520x; 1.0520x over previous
import functools

import jax
import jax.numpy as jnp
from jax import lax
from jax.experimental import pallas as pl
from jax.experimental.pallas import tpu as pltpu
from jax.experimental.pallas import tpu_sc as plsc

N = 10000
E = 320000
HID = 128
OUT_DIM = 40
NUM_LAYERS = 7
EPS = 1e-7
BN_EPS = 1e-5

NT = 16
EB = 112
CH = 3
NJ = -(-E // (NT * EB * CH))
NJ = 3 * (-(-NJ // 3))
NB = NJ * CH
E_PAD = NT * NB * EB
ACC_ROWS = 10240
ZR = ACC_ROWS // NT
RB = 2000


def _sc_aggregate(tg, tp, ix_p, zeros_blk):
    mesh = plsc.VectorSubcoreMesh(core_axis_name="c", subcore_axis_name="s")

    @functools.partial(
        pl.kernel,
        out_type=(jax.ShapeDtypeStruct((ACC_ROWS, HID), jnp.float32),
                  jax.ShapeDtypeStruct((ACC_ROWS, HID), jnp.float32)),
        mesh=mesh,
        scratch_types=(
            [pltpu.VMEM((2, CH, EB), jnp.int32)] * 3
            + [pltpu.VMEM((EB, HID), jnp.float32)] * 3
            + [pltpu.VMEM_SHARED((ACC_ROWS, HID), jnp.float32)]
            + [pltpu.SemaphoreType.DMA] * 9
        ),
    )
    def k(tg_hbm, tp_hbm, ix_hbm, z_hbm, den_out, num_out,
          ib0, ib1, ib2, rows0, rows1, rows2, acc,
          csem0, csem1, csem2, gsem0, gsem1, gsem2, ssem0, ssem1, ssem2):
        c = lax.axis_index("c")
        s = lax.axis_index("s")

        IB = [ib0, ib1, ib2]
        RW = [rows0, rows1, rows2]
        CS = [csem0, csem1, csem2]
        GS = [gsem0, gsem1, gsem2]
        SS = [ssem0, ssem1, ssem2]

        pltpu.sync_copy(z_hbm, acc.at[pl.ds(s * ZR, ZR)])
        plsc.subcore_barrier()

        def run(t_hbm):
            def chunk_fetch(kc, q):
                pltpu.async_copy(ix_hbm.at[s, kc], IB[q], CS[q])

            def chunk_wait(kc, q):
                pltpu.make_async_copy(ix_hbm.at[s, kc], IB[q], CS[q]).wait()

            def gather_start(r, sref):
                pltpu.async_copy(t_hbm.at[sref], RW[r], GS[r])

            def gather_wait(r, sref):
                pltpu.make_async_copy(t_hbm.at[sref], RW[r], GS[r]).wait()

            def scatter_start(r, dref):
                pltpu.async_copy(RW[r], acc.at[dref], SS[r], add=True)

            def scatter_wait(r, dref):
                pltpu.make_async_copy(RW[r], acc.at[dref], SS[r]).wait()

            def visit(kc, q, v, first=False, fetch=True, wait_next=True,
                      gnext=True):
                r = v
                rp = (v + 2) % 3
                qn = (q + 1) % 3
                qp = (q + 2) % 3
                gather_wait(r, IB[q].at[0, v])
                if not first:
                    if v == 0:
                        scatter_wait(rp, IB[qp].at[1, 2])
                    else:
                        scatter_wait(rp, IB[q].at[1, v - 1])
                if v == 0 and fetch:
                    chunk_fetch(kc + 2, qp)
                if v == 1 and wait_next:
                    chunk_wait(kc + 1, qn)
                scatter_start(r, IB[q].at[1, v])
                if gnext:
                    if v == 0:
                        gather_start(rp, IB[q].at[0, 2])
                    else:
                        gather_start(rp, IB[qn].at[0, v - 1])

            chunk_fetch(0, 0)
            chunk_fetch(1, 1)
            chunk_wait(0, 0)
            gather_start(0, IB[0].at[0, 0])
            gather_start(1, IB[0].at[0, 1])

            visit(0, 0, 0, first=True)
            visit(0, 0, 1)
            visit(0, 0, 2)
            for kk in (1, 2):
                visit(kk, kk, 0)
                visit(kk, kk, 1)
                visit(kk, kk, 2)

            def body(m, carry):
                k0 = 3 * m
                for q in (0, 1, 2):
                    visit(k0 + q, q, 0)
                    visit(k0 + q, q, 1)
                    visit(k0 + q, q, 2)
                return carry

            lax.fori_loop(1, NJ // 3 - 1, body, 0)

            visit(NJ - 3, 0, 0)
            visit(NJ - 3, 0, 1)
            visit(NJ - 3, 0, 2)
            visit(NJ - 2, 1, 0, fetch=False)
            visit(NJ - 2, 1, 1)
            visit(NJ - 2, 1, 2)
            visit(NJ - 1, 2, 0, fetch=False)
            visit(NJ - 1, 2, 1, wait_next=False, gnext=False)
            visit(NJ - 1, 2, 2, gnext=False)
            scatter_wait(2, IB[2].at[1, 2])

        @pl.when(c == 0)
        def _():
            run(tg_hbm)

        @pl.when(c == 1)
        def _():
            run(tp_hbm)

        plsc.subcore_barrier()

        @pl.when(c == 0)
        def _():
            pltpu.sync_copy(acc.at[pl.ds(s * ZR, ZR)],
                            den_out.at[pl.ds(s * ZR, ZR)])

        @pl.when(c == 1)
        def _():
            pltpu.sync_copy(acc.at[pl.ds(s * ZR, ZR)],
                            num_out.at[pl.ds(s * ZR, ZR)])

    den, num = k(tg, tp, ix_p, zeros_blk)
    return den[:N], num[:N]


def _bn_tables(h, gamma, beta, hn_ref, tg_ref, tp_ref):
    mean = jnp.mean(h, axis=0, keepdims=True)
    d = h - mean
    var = jnp.mean(d * d, axis=0, keepdims=True)
    hn = d * lax.rsqrt(var + BN_EPS) * gamma + beta
    hn = jnp.maximum(hn, 0.0)
    g = hn + EPS
    cmax = jnp.max(g, axis=0, keepdims=True)
    tg = jnp.exp(g - cmax)
    hn_ref[...] = hn
    tg_ref[...] = tg
    tp_ref[...] = g * tg


def _enc_pre_body(x_ref, w_ref, b_ref, g_ref, be_ref,
                  h_ref, hn_ref, tg_ref, tp_ref):
    h = jnp.dot(x_ref[...], w_ref[...],
                preferred_element_type=jnp.float32) + b_ref[...]
    h_ref[...] = h
    _bn_tables(h, g_ref[...], be_ref[...], hn_ref, tg_ref, tp_ref)


def _enc_pre(x, w, b, gamma, beta):
    spec = pl.BlockSpec((N, HID), lambda: (0, 0))
    vspec = pl.BlockSpec((1, HID), lambda: (0, 0))
    wspec = pl.BlockSpec((HID, HID), lambda: (0, 0))
    return pl.pallas_call(
        _enc_pre_body,
        in_specs=[spec, wspec, vspec, vspec, vspec],
        out_specs=(spec, spec, spec, spec),
        out_shape=(jax.ShapeDtypeStruct((N, HID), jnp.float32),) * 4,
    )(x, w, b, gamma, beta)


def _conv_out(hn_ref, num_ref, den_ref, h_ref, w_ref, b_ref):
    msg = num_ref[...] / (den_ref[...] + 1e-16)
    l2 = jnp.sqrt(jnp.sum(msg * msg, axis=1, keepdims=True))
    hn = hn_ref[...]
    fn = jnp.sqrt(jnp.sum(hn * hn, axis=1, keepdims=True))
    msg = msg / jnp.maximum(l2, 1e-12) * fn
    feats = hn + msg
    return h_ref[...] + jnp.dot(feats, w_ref[...],
                                preferred_element_type=jnp.float32) + b_ref[...]


def _post_pre_body(hn_ref, num_ref, den_ref, h_ref, w_ref, b_ref,
                   g_ref, be_ref, h2_ref, hn2_ref, tg_ref, tp_ref):
    h2 = _conv_out(hn_ref, num_ref, den_ref, h_ref, w_ref, b_ref)
    h2_ref[...] = h2
    _bn_tables(h2, g_ref[...], be_ref[...], hn2_ref, tg_ref, tp_ref)


def _post_pre(hn, num, den, h, w, b, gamma, beta):
    spec = pl.BlockSpec((N, HID), lambda: (0, 0))
    vspec = pl.BlockSpec((1, HID), lambda: (0, 0))
    wspec = pl.BlockSpec((HID, HID), lambda: (0, 0))
    return pl.pallas_call(
        _post_pre_body,
        in_specs=[spec, spec, spec, spec, wspec, vspec, vspec, vspec],
        out_specs=(spec, spec, spec, spec),
        out_shape=(jax.ShapeDtypeStruct((N, HID), jnp.float32),) * 4,
    )(hn, num, den, h, w, b, gamma, beta)


def _post_final_body(hn_ref, num_ref, den_ref, h_ref, w_ref, b_ref,
                     wp_ref, bp_ref, o_ref):
    hlast = _conv_out(hn_ref, num_ref, den_ref, h_ref, w_ref, b_ref)
    z = jnp.dot(hlast, wp_ref[...],
                preferred_element_type=jnp.float32) + bp_ref[...]
    col = lax.broadcasted_iota(jnp.int32, z.shape, 1)
    valid = col < OUT_DIM
    zm = jnp.where(valid, z, -jnp.inf)
    mx = jnp.max(zm, axis=1, keepdims=True)
    e = jnp.where(valid, jnp.exp(zm - mx), 0.0)
    lse = jnp.log(jnp.sum(e, axis=1, keepdims=True))
    out = zm - mx - lse
    o_ref[...] = out[:, :OUT_DIM]


def _post_final(hn, num, den, h, w, b, wp, bp):
    spec = pl.BlockSpec((N, HID), lambda: (0, 0))
    vspec = pl.BlockSpec((1, HID), lambda: (0, 0))
    wspec = pl.BlockSpec((HID, HID), lambda: (0, 0))
    return pl.pallas_call(
        _post_final_body,
        in_specs=[spec, spec, spec, spec, wspec, vspec, wspec, vspec],
        out_specs=pl.BlockSpec((N, OUT_DIM), lambda: (0, 0)),
        out_shape=jax.ShapeDtypeStruct((N, OUT_DIM), jnp.float32),
    )(hn, num, den, h, w, b, wp, bp)


def kernel(node_feats, edge_index, W_enc, b_enc, W_conv, b_conv,
           bn_gamma, bn_beta, W_out, b_out):
    src = edge_index[0]
    dst = edge_index[1]
    pad = E_PAD - E
    src_p = jnp.concatenate([src, jnp.zeros((pad,), jnp.int32)]).reshape(
        NT, NJ, CH, EB)
    dst_p = jnp.concatenate([dst, jnp.full((pad,), N, jnp.int32)]).reshape(
        NT, NJ, CH, EB)
    ix_p = jnp.stack([src_p, dst_p], axis=2)
    zeros_blk = jnp.zeros((ZR, HID), jnp.float32)

    h, hn, tg, tp = _enc_pre(node_feats, W_enc, b_enc.reshape(1, HID),
                             bn_gamma[0].reshape(1, HID),
                             bn_beta[0].reshape(1, HID))
    for i in range(NUM_LAYERS - 1):
        den, num = _sc_aggregate(tg, tp, ix_p, zeros_blk)
        h, hn, tg, tp = _post_pre(hn, num, den, h, W_conv[i],
                                  b_conv[i].reshape(1, HID),
                                  bn_gamma[i + 1].reshape(1, HID),
                                  bn_beta[i + 1].reshape(1, HID))
    den, num = _sc_aggregate(tg, tp, ix_p, zeros_blk)

    wp = jnp.pad(W_out, ((0, 0), (0, HID - OUT_DIM)))
    bp = jnp.pad(b_out, (0, HID - OUT_DIM)).reshape(1, HID)
    return _post_final(hn, num, den, h, W_conv[NUM_LAYERS - 1],
                       b_conv[NUM_LAYERS - 1].reshape(1, HID), wp, bp)
